# Initial kernel scaffold; baseline (speedup 1.0000x reference)
import jax
import jax.numpy as jnp
from jax.experimental import pallas as pl


def kernel(x, edge_index, W1, b1, W2, b2, W3, b3):
    raise NotImplementedError("write your pallas kernel here")


# speedup vs baseline: 12.1999x; 12.1999x over previous
import functools

import jax
import jax.numpy as jnp
from jax import lax
from jax.experimental import pallas as pl
from jax.experimental.pallas import tpu as pltpu
from jax.experimental.pallas import tpu_sc as plsc

N = 10000
E = 320000
NC, NS = 2, 16
EPC = E // NC
EPT = EPC // NS
K = 80
NCHUNK = EPT // K
NPAD = 10240
RPT = NPAD // NS
ZR = 128
DEGW = 16

_MESH = dict(core_axis_name="c", subcore_axis_name="s")


def _deg_partials(dst):
    @functools.partial(
        pl.kernel,
        out_type=jax.ShapeDtypeStruct((NC, NPAD, DEGW), jnp.float32),
        mesh=plsc.VectorSubcoreMesh(**_MESH),
        compiler_params=pltpu.CompilerParams(use_tc_tiling_on_sc=False),
        scratch_types=[
            pltpu.VMEM((K,), jnp.int32),
            pltpu.VMEM((K, DEGW), jnp.float32),
            pltpu.VMEM((ZR, DEGW), jnp.float32),
            pltpu.VMEM_SHARED((NPAD, DEGW), jnp.float32),
        ],
    )
    def deg_k(dst_hbm, out_hbm, idx_v, ones_v, zbuf_v, acc_sh):
        c = lax.axis_index("c")
        s = lax.axis_index("s")

        def fill_ones(i, carry):
            ones_v[i, :] = jnp.ones((DEGW,), jnp.float32)
            return carry

        lax.fori_loop(0, K, fill_ones, 0)

        def fill_zero(i, carry):
            zbuf_v[i, :] = jnp.zeros((DEGW,), jnp.float32)
            return carry

        lax.fori_loop(0, ZR, fill_zero, 0)

        row0 = s * RPT
        for z in range(RPT // ZR):
            pltpu.sync_copy(zbuf_v, acc_sh.at[pl.ds(row0 + z * ZR, ZR)])
        plsc.subcore_barrier()

        ebase = c * EPC + s * EPT

        def body(i, carry):
            off = pl.multiple_of(ebase + i * K, 8)
            pltpu.sync_copy(dst_hbm.at[pl.ds(off, K)], idx_v)
            pltpu.sync_copy(ones_v, acc_sh.at[idx_v], add=True)
            return carry

        lax.fori_loop(0, NCHUNK, body, 0)
        plsc.subcore_barrier()
        pltpu.sync_copy(acc_sh.at[pl.ds(row0, RPT)],
                        out_hbm.at[c, pl.ds(row0, RPT)])

    return deg_k(dst)


@functools.lru_cache(maxsize=None)
def _make_segsum(D):
    @functools.partial(
        pl.kernel,
        out_type=jax.ShapeDtypeStruct((NC, NPAD, D), jnp.float32),
        mesh=plsc.VectorSubcoreMesh(**_MESH),
        compiler_params=pltpu.CompilerParams(use_tc_tiling_on_sc=False),
        scratch_types=[
            pltpu.VMEM((K,), jnp.int32),
            pltpu.VMEM((K,), jnp.int32),
            pltpu.VMEM((K, D), jnp.float32),
            pltpu.VMEM((ZR, D), jnp.float32),
            pltpu.VMEM_SHARED((NPAD, D), jnp.float32),
            pltpu.SemaphoreType.DMA,
        ],
    )
    def seg_k(g_hbm, src_hbm, dst_hbm, out_hbm,
              sidx_v, didx_v, rows_v, zbuf_v, acc_sh, sem):
        c = lax.axis_index("c")
        s = lax.axis_index("s")

        def fill_zero(i, carry):
            def inner(j, carry2):
                zbuf_v[i, pl.ds(j * 16, 16)] = jnp.zeros((16,), jnp.float32)
                return carry2
            return lax.fori_loop(0, D // 16, inner, carry)

        lax.fori_loop(0, ZR, fill_zero, 0)

        row0 = s * RPT
        for z in range(RPT // ZR):
            pltpu.sync_copy(zbuf_v, acc_sh.at[pl.ds(row0 + z * ZR, ZR)])
        plsc.subcore_barrier()

        ebase = c * EPC + s * EPT

        def body(i, carry):
            off = pl.multiple_of(ebase + i * K, 8)
            pltpu.sync_copy(src_hbm.at[pl.ds(off, K)], sidx_v)
            pltpu.sync_copy(dst_hbm.at[pl.ds(off, K)], didx_v)
            pltpu.async_copy(g_hbm.at[sidx_v], rows_v, sem).wait()
            pltpu.sync_copy(rows_v, acc_sh.at[didx_v], add=True)
            return carry

        lax.fori_loop(0, NCHUNK, body, 0)
        plsc.subcore_barrier()
        pltpu.sync_copy(acc_sh.at[pl.ds(row0, RPT)],
                        out_hbm.at[c, pl.ds(row0, RPT)])

    return seg_k


def _dinv(degp_ref):
    deg = degp_ref[0, :N, :] + degp_ref[1, :N, :]
    return lax.rsqrt(deg[:, 0:1] + 1.0)


def _dot(a, w):
    return jnp.dot(a, w, preferred_element_type=jnp.float32,
                   precision=lax.Precision.HIGHEST)


def _tc_pre(x, W1, degp):
    def body(x_ref, w_ref, degp_ref, out_ref):
        out_ref[...] = _dinv(degp_ref) * _dot(x_ref[...], w_ref[...])

    return pl.pallas_call(
        body,
        out_shape=jax.ShapeDtypeStruct((N, W1.shape[1]), jnp.float32),
    )(x, W1, degp)


def _tc_mid(pp, g, degp, b, Wn):
    def body(pp_ref, g_ref, degp_ref, b_ref, w_ref, out_ref):
        dinv = _dinv(degp_ref)
        h = dinv * (pp_ref[0, :N, :] + pp_ref[1, :N, :] + g_ref[...]) + b_ref[...]
        h = jnp.maximum(h, 0.0)
        out_ref[...] = dinv * _dot(h, w_ref[...])

    return pl.pallas_call(
        body,
        out_shape=jax.ShapeDtypeStruct((N, Wn.shape[1]), jnp.float32),
    )(pp, g, degp, b, Wn)


def _tc_fin(pp, g, degp, b):
    def body(pp_ref, g_ref, degp_ref, b_ref, out_ref):
        dinv = _dinv(degp_ref)
        out_ref[...] = dinv * (pp_ref[0, :N, :] + pp_ref[1, :N, :] + g_ref[...]) + b_ref[...]

    return pl.pallas_call(
        body,
        out_shape=jax.ShapeDtypeStruct((N, g.shape[1]), jnp.float32),
    )(pp, g, degp, b)


def kernel(x, edge_index, W1, b1, W2, b2, W3, b3):
    ei = edge_index.astype(jnp.int32)
    src, dst = ei[0], ei[1]

    degp = _deg_partials(dst)

    g1 = _tc_pre(x, W1, degp)
    pp1 = _make_segsum(128)(g1, src, dst)
    g2 = _tc_mid(pp1, g1, degp, b1.reshape(1, -1), W2)
    pp2 = _make_segsum(64)(g2, src, dst)
    g3 = _tc_mid(pp2, g2, degp, b2.reshape(1, -1), W3)
    pp3 = _make_segsum(64)(g3, src, dst)
    return _tc_fin(pp3, g3, degp, b3.reshape(1, -1))


# speedup vs baseline: 25.6057x; 2.0988x over previous
import functools

import jax
import jax.numpy as jnp
from jax import lax
from jax.experimental import pallas as pl
from jax.experimental.pallas import tpu as pltpu
from jax.experimental.pallas import tpu_sc as plsc

N = 10000
E = 320000
NC, NS = 2, 16
EPC = E // NC
EPT = EPC // NS
K = 80
NCHUNK = EPT // K
NPAD = 10240
RPT = NPAD // NS
ZR = 128
DEGW = 16

_MESH = dict(core_axis_name="c", subcore_axis_name="s")


def _deg_partials(dst):
    @functools.partial(
        pl.kernel,
        out_type=jax.ShapeDtypeStruct((NC, NPAD, DEGW), jnp.float32),
        mesh=plsc.VectorSubcoreMesh(**_MESH),
        compiler_params=pltpu.CompilerParams(use_tc_tiling_on_sc=False),
        scratch_types=[
            pltpu.VMEM((K,), jnp.int32),
            pltpu.VMEM((K, DEGW), jnp.float32),
            pltpu.VMEM((ZR, DEGW), jnp.float32),
            pltpu.VMEM_SHARED((NPAD, DEGW), jnp.float32),
        ],
    )
    def deg_k(dst_hbm, out_hbm, idx_v, ones_v, zbuf_v, acc_sh):
        c = lax.axis_index("c")
        s = lax.axis_index("s")

        def fill_ones(i, carry):
            ones_v[i, :] = jnp.ones((DEGW,), jnp.float32)
            return carry

        lax.fori_loop(0, K, fill_ones, 0)

        def fill_zero(i, carry):
            zbuf_v[i, :] = jnp.zeros((DEGW,), jnp.float32)
            return carry

        lax.fori_loop(0, ZR, fill_zero, 0)

        row0 = s * RPT
        for z in range(RPT // ZR):
            pltpu.sync_copy(zbuf_v, acc_sh.at[pl.ds(row0 + z * ZR, ZR)])
        plsc.subcore_barrier()

        ebase = c * EPC + s * EPT

        def body(i, carry):
            off = pl.multiple_of(ebase + i * K, 8)
            pltpu.sync_copy(dst_hbm.at[pl.ds(off, K)], idx_v)
            pltpu.sync_copy(ones_v, acc_sh.at[idx_v], add=True)
            return carry

        lax.fori_loop(0, NCHUNK, body, 0)
        plsc.subcore_barrier()
        pltpu.sync_copy(acc_sh.at[pl.ds(row0, RPT)],
                        out_hbm.at[c, pl.ds(row0, RPT)])

    return deg_k(dst)


@functools.lru_cache(maxsize=None)
def _make_segsum(D):
    @functools.partial(
        pl.kernel,
        out_type=jax.ShapeDtypeStruct((NC, NPAD, D), jnp.float32),
        mesh=plsc.VectorSubcoreMesh(**_MESH),
        compiler_params=pltpu.CompilerParams(use_tc_tiling_on_sc=False),
        scratch_types=[
            pltpu.VMEM((NCHUNK, K), jnp.int32),
            pltpu.VMEM((NCHUNK, K), jnp.int32),
            pltpu.VMEM((K, D), jnp.float32),
            pltpu.VMEM((K, D), jnp.float32),
            pltpu.VMEM_SHARED((NPAD, D), jnp.float32),
            pltpu.SemaphoreType.DMA,
            pltpu.SemaphoreType.DMA,
        ],
    )
    def seg_k(g_hbm, src2_hbm, dst2_hbm, out_hbm,
              sidx_v, didx_v, rows_a, rows_b, acc_sh, sem_a, sem_b):
        c = lax.axis_index("c")
        s = lax.axis_index("s")

        def fill_zero(i, carry):
            def inner(j, carry2):
                rows_a[i, pl.ds(j * 16, 16)] = jnp.zeros((16,), jnp.float32)
                return carry2
            return lax.fori_loop(0, D // 16, inner, carry)

        lax.fori_loop(0, K, fill_zero, 0)

        cbase = c * (EPC // K) + s * NCHUNK
        pltpu.sync_copy(src2_hbm.at[pl.ds(cbase, NCHUNK)], sidx_v)
        pltpu.sync_copy(dst2_hbm.at[pl.ds(cbase, NCHUNK)], didx_v)

        row0 = s * RPT
        for z in range(RPT // K):
            pltpu.sync_copy(rows_a, acc_sh.at[pl.ds(row0 + z * K, K)])
        plsc.subcore_barrier()

        def gather(i, buf, sem):
            return pltpu.async_copy(g_hbm.at[sidx_v.at[i]], buf, sem)

        def scatter(i, buf):
            pltpu.sync_copy(buf, acc_sh.at[didx_v.at[i]], add=True)

        gather(0, rows_a, sem_a)

        def body(p, carry):
            i0 = p * 2
            db = gather(i0 + 1, rows_b, sem_b)
            pltpu.make_async_copy(g_hbm.at[sidx_v.at[i0]], rows_a, sem_a).wait()
            scatter(i0, rows_a)
            da = gather(i0 + 2, rows_a, sem_a)
            db.wait()
            scatter(i0 + 1, rows_b)
            return carry

        lax.fori_loop(0, (NCHUNK - 1) // 2, body, 0)
        pltpu.make_async_copy(
            g_hbm.at[sidx_v.at[NCHUNK - 1]], rows_a, sem_a).wait()
        scatter(NCHUNK - 1, rows_a)

        plsc.subcore_barrier()
        pltpu.sync_copy(acc_sh.at[pl.ds(row0, RPT)],
                        out_hbm.at[c, pl.ds(row0, RPT)])

    return seg_k


def _dinv(degp_ref):
    deg = degp_ref[0, :N, :] + degp_ref[1, :N, :]
    return lax.rsqrt(deg[:, 0:1] + 1.0)


def _dot(a, w):
    return jnp.dot(a, w, preferred_element_type=jnp.float32,
                   precision=lax.Precision.HIGHEST)


def _tc_pre(x, W1, degp):
    def body(x_ref, w_ref, degp_ref, out_ref):
        out_ref[...] = _dinv(degp_ref) * _dot(x_ref[...], w_ref[...])

    return pl.pallas_call(
        body,
        out_shape=jax.ShapeDtypeStruct((N, W1.shape[1]), jnp.float32),
    )(x, W1, degp)


def _tc_mid(pp, g, degp, b, Wn):
    def body(pp_ref, g_ref, degp_ref, b_ref, w_ref, out_ref):
        dinv = _dinv(degp_ref)
        h = dinv * (pp_ref[0, :N, :] + pp_ref[1, :N, :] + g_ref[...]) + b_ref[...]
        h = jnp.maximum(h, 0.0)
        out_ref[...] = dinv * _dot(h, w_ref[...])

    return pl.pallas_call(
        body,
        out_shape=jax.ShapeDtypeStruct((N, Wn.shape[1]), jnp.float32),
    )(pp, g, degp, b, Wn)


def _tc_fin(pp, g, degp, b):
    def body(pp_ref, g_ref, degp_ref, b_ref, out_ref):
        dinv = _dinv(degp_ref)
        out_ref[...] = dinv * (pp_ref[0, :N, :] + pp_ref[1, :N, :] + g_ref[...]) + b_ref[...]

    return pl.pallas_call(
        body,
        out_shape=jax.ShapeDtypeStruct((N, g.shape[1]), jnp.float32),
    )(pp, g, degp, b)


def kernel(x, edge_index, W1, b1, W2, b2, W3, b3):
    ei = edge_index.astype(jnp.int32)
    src, dst = ei[0], ei[1]

    degp = _deg_partials(dst)
    src2 = src.reshape(E // K, K)
    dst2 = dst.reshape(E // K, K)

    g1 = _tc_pre(x, W1, degp)
    pp1 = _make_segsum(128)(g1, src2, dst2)
    g2 = _tc_mid(pp1, g1, degp, b1.reshape(1, -1), W2)
    pp2 = _make_segsum(64)(g2, src2, dst2)
    g3 = _tc_mid(pp2, g2, degp, b2.reshape(1, -1), W3)
    pp3 = _make_segsum(64)(g3, src2, dst2)
    return _tc_fin(pp3, g3, degp, b3.reshape(1, -1))


# speedup vs baseline: 29.2051x; 1.1406x over previous
import functools

import jax
import jax.numpy as jnp
from jax import lax
from jax.experimental import pallas as pl
from jax.experimental.pallas import tpu as pltpu
from jax.experimental.pallas import tpu_sc as plsc

N = 10000
E = 320000
NC, NS = 2, 16
EPC = E // NC
EPT = EPC // NS
K = 80
NCHUNK = EPT // K
NPAD = 10240
RPT = NPAD // NS
ZR = 128
DEGW = 16

_MESH = dict(core_axis_name="c", subcore_axis_name="s")


def _deg_partials(dst):
    @functools.partial(
        pl.kernel,
        out_type=jax.ShapeDtypeStruct((NC, NPAD, DEGW), jnp.float32),
        mesh=plsc.VectorSubcoreMesh(**_MESH),
        compiler_params=pltpu.CompilerParams(use_tc_tiling_on_sc=False),
        scratch_types=[
            pltpu.VMEM((NCHUNK, K), jnp.int32),
            pltpu.VMEM((K, DEGW), jnp.float32),
            pltpu.VMEM_SHARED((NPAD, DEGW), jnp.float32),
            pltpu.SemaphoreType.DMA,
        ],
    )
    def deg_k(dst2_hbm, out_hbm, didx_v, ones_v, acc_sh, sem):
        c = lax.axis_index("c")
        s = lax.axis_index("s")

        def fill_ones(i, carry):
            ones_v[i, :] = jnp.ones((DEGW,), jnp.float32)
            return carry

        lax.fori_loop(0, K, fill_ones, 0)

        cbase = c * (EPC // K) + s * NCHUNK
        pltpu.sync_copy(dst2_hbm.at[pl.ds(cbase, NCHUNK)], didx_v)

        row0 = s * RPT

        def fill_zero(i, carry):
            ones_v[i, :] = jnp.zeros((DEGW,), jnp.float32)
            return carry

        lax.fori_loop(0, K, fill_zero, 0)
        for z in range(RPT // K):
            pltpu.sync_copy(ones_v, acc_sh.at[pl.ds(row0 + z * K, K)])
        lax.fori_loop(0, K, fill_ones, 0)
        plsc.subcore_barrier()

        def body(i, carry):
            pltpu.async_copy(ones_v, acc_sh.at[didx_v.at[i]], sem, add=True)
            return carry

        lax.fori_loop(0, NCHUNK, body, 0)

        def drain(i, carry):
            pltpu.make_async_copy(ones_v, acc_sh.at[didx_v.at[i]], sem).wait()
            return carry

        lax.fori_loop(0, NCHUNK, drain, 0)
        plsc.subcore_barrier()
        pltpu.sync_copy(acc_sh.at[pl.ds(row0, RPT)],
                        out_hbm.at[c, pl.ds(row0, RPT)])

    return deg_k(dst)


@functools.lru_cache(maxsize=None)
def _make_segsum(D):
    @functools.partial(
        pl.kernel,
        out_type=jax.ShapeDtypeStruct((NC, NPAD, D), jnp.float32),
        mesh=plsc.VectorSubcoreMesh(**_MESH),
        compiler_params=pltpu.CompilerParams(use_tc_tiling_on_sc=False),
        scratch_types=[
            pltpu.VMEM((NCHUNK, K), jnp.int32),
            pltpu.VMEM((NCHUNK, K), jnp.int32),
            pltpu.VMEM((K, D), jnp.float32),
            pltpu.VMEM((K, D), jnp.float32),
            pltpu.VMEM_SHARED((NPAD, D), jnp.float32),
            pltpu.SemaphoreType.DMA,
            pltpu.SemaphoreType.DMA,
        ],
    )
    def seg_k(g_hbm, src2_hbm, dst2_hbm, out_hbm,
              sidx_v, didx_v, rows_a, rows_b, acc_sh, sem_a, sem_b):
        c = lax.axis_index("c")
        s = lax.axis_index("s")

        def fill_zero(i, carry):
            def inner(j, carry2):
                rows_a[i, pl.ds(j * 16, 16)] = jnp.zeros((16,), jnp.float32)
                return carry2
            return lax.fori_loop(0, D // 16, inner, carry)

        lax.fori_loop(0, K, fill_zero, 0)

        cbase = c * (EPC // K) + s * NCHUNK
        pltpu.sync_copy(src2_hbm.at[pl.ds(cbase, NCHUNK)], sidx_v)
        pltpu.sync_copy(dst2_hbm.at[pl.ds(cbase, NCHUNK)], didx_v)

        row0 = s * RPT
        for z in range(RPT // K):
            pltpu.sync_copy(rows_a, acc_sh.at[pl.ds(row0 + z * K, K)])
        plsc.subcore_barrier()

        def gather(i, buf, sem):
            return pltpu.async_copy(g_hbm.at[sidx_v.at[i]], buf, sem)

        def scatter(i, buf):
            pltpu.sync_copy(buf, acc_sh.at[didx_v.at[i]], add=True)

        gather(0, rows_a, sem_a)

        def body(p, carry):
            i0 = p * 2
            db = gather(i0 + 1, rows_b, sem_b)
            pltpu.make_async_copy(g_hbm.at[sidx_v.at[i0]], rows_a, sem_a).wait()
            scatter(i0, rows_a)
            da = gather(i0 + 2, rows_a, sem_a)
            db.wait()
            scatter(i0 + 1, rows_b)
            return carry

        lax.fori_loop(0, (NCHUNK - 1) // 2, body, 0)
        pltpu.make_async_copy(
            g_hbm.at[sidx_v.at[NCHUNK - 1]], rows_a, sem_a).wait()
        scatter(NCHUNK - 1, rows_a)

        plsc.subcore_barrier()
        pltpu.sync_copy(acc_sh.at[pl.ds(row0, RPT)],
                        out_hbm.at[c, pl.ds(row0, RPT)])

    return seg_k


def _dinv(degp_ref):
    deg = degp_ref[0, :N, :] + degp_ref[1, :N, :]
    return lax.rsqrt(deg[:, 0:1] + 1.0)


def _dot(a, w):
    return jnp.dot(a, w, preferred_element_type=jnp.float32,
                   precision=lax.Precision.HIGHEST)


def _tc_pre(x, W1, degp):
    def body(x_ref, w_ref, degp_ref, out_ref):
        out_ref[...] = _dinv(degp_ref) * _dot(x_ref[...], w_ref[...])

    return pl.pallas_call(
        body,
        out_shape=jax.ShapeDtypeStruct((N, W1.shape[1]), jnp.float32),
    )(x, W1, degp)


def _tc_mid(pp, g, degp, b, Wn):
    def body(pp_ref, g_ref, degp_ref, b_ref, w_ref, out_ref):
        dinv = _dinv(degp_ref)
        h = dinv * (pp_ref[0, :N, :] + pp_ref[1, :N, :] + g_ref[...]) + b_ref[...]
        h = jnp.maximum(h, 0.0)
        out_ref[...] = dinv * _dot(h, w_ref[...])

    return pl.pallas_call(
        body,
        out_shape=jax.ShapeDtypeStruct((N, Wn.shape[1]), jnp.float32),
    )(pp, g, degp, b, Wn)


def _tc_fin(pp, g, degp, b):
    def body(pp_ref, g_ref, degp_ref, b_ref, out_ref):
        dinv = _dinv(degp_ref)
        out_ref[...] = dinv * (pp_ref[0, :N, :] + pp_ref[1, :N, :] + g_ref[...]) + b_ref[...]

    return pl.pallas_call(
        body,
        out_shape=jax.ShapeDtypeStruct((N, g.shape[1]), jnp.float32),
    )(pp, g, degp, b)


def kernel(x, edge_index, W1, b1, W2, b2, W3, b3):
    ei = edge_index.astype(jnp.int32)
    src, dst = ei[0], ei[1]

    src2 = src.reshape(E // K, K)
    dst2 = dst.reshape(E // K, K)
    degp = _deg_partials(dst2)

    g1 = _tc_pre(x, W1, degp)
    pp1 = _make_segsum(128)(g1, src2, dst2)
    g2 = _tc_mid(pp1, g1, degp, b1.reshape(1, -1), W2)
    pp2 = _make_segsum(64)(g2, src2, dst2)
    g3 = _tc_mid(pp2, g2, degp, b2.reshape(1, -1), W3)
    pp3 = _make_segsum(64)(g3, src2, dst2)
    return _tc_fin(pp3, g3, degp, b3.reshape(1, -1))


# speedup vs baseline: 32.7276x; 1.1206x over previous
import functools

import jax
import jax.numpy as jnp
from jax import lax
from jax.experimental import pallas as pl
from jax.experimental.pallas import tpu as pltpu
from jax.experimental.pallas import tpu_sc as plsc

N = 10000
E = 320000
NC, NS = 2, 16
EPC = E // NC
EPT = EPC // NS
K = 80
NCHUNK = EPT // K
NPAD = 10240
RPT = NPAD // NS
ZR = 128
DEGW = 16

_MESH = dict(core_axis_name="c", subcore_axis_name="s")


def _deg_partials(dst):
    @functools.partial(
        pl.kernel,
        out_type=jax.ShapeDtypeStruct((NC, NPAD, DEGW), jnp.float32),
        mesh=plsc.VectorSubcoreMesh(**_MESH),
        compiler_params=pltpu.CompilerParams(use_tc_tiling_on_sc=False),
        scratch_types=[
            pltpu.VMEM((NCHUNK, K), jnp.int32),
            pltpu.VMEM((K, DEGW), jnp.float32),
            pltpu.VMEM_SHARED((NPAD, DEGW), jnp.float32),
            pltpu.SemaphoreType.DMA,
        ],
    )
    def deg_k(dst2_hbm, out_hbm, didx_v, ones_v, acc_sh, sem):
        c = lax.axis_index("c")
        s = lax.axis_index("s")

        def fill_ones(i, carry):
            ones_v[i, :] = jnp.ones((DEGW,), jnp.float32)
            return carry

        lax.fori_loop(0, K, fill_ones, 0)

        cbase = c * (EPC // K) + s * NCHUNK
        pltpu.sync_copy(dst2_hbm.at[pl.ds(cbase, NCHUNK)], didx_v)

        row0 = s * RPT

        def fill_zero(i, carry):
            ones_v[i, :] = jnp.zeros((DEGW,), jnp.float32)
            return carry

        lax.fori_loop(0, K, fill_zero, 0)
        for z in range(RPT // K):
            pltpu.sync_copy(ones_v, acc_sh.at[pl.ds(row0 + z * K, K)])
        lax.fori_loop(0, K, fill_ones, 0)
        plsc.subcore_barrier()

        def body(i, carry):
            pltpu.async_copy(ones_v, acc_sh.at[didx_v.at[i]], sem, add=True)
            return carry

        lax.fori_loop(0, NCHUNK, body, 0)

        def drain(i, carry):
            pltpu.make_async_copy(ones_v, acc_sh.at[didx_v.at[i]], sem).wait()
            return carry

        lax.fori_loop(0, NCHUNK, drain, 0)
        plsc.subcore_barrier()
        pltpu.sync_copy(acc_sh.at[pl.ds(row0, RPT)],
                        out_hbm.at[c, pl.ds(row0, RPT)])

    return deg_k(dst)


@functools.lru_cache(maxsize=None)
def _make_segsum(D):
    SRPT = 624

    @functools.partial(
        pl.kernel,
        out_type=jax.ShapeDtypeStruct((NC, N, D), jnp.float32),
        mesh=plsc.VectorSubcoreMesh(**_MESH),
        compiler_params=pltpu.CompilerParams(use_tc_tiling_on_sc=False),
        scratch_types=[
            pltpu.VMEM((NCHUNK, K), jnp.int32),
            pltpu.VMEM((NCHUNK, K), jnp.int32),
            pltpu.VMEM((K, D), jnp.float32),
            pltpu.VMEM((K, D), jnp.float32),
            pltpu.VMEM((K, D), jnp.float32),
            pltpu.VMEM_SHARED((N, D), jnp.float32),
            pltpu.SemaphoreType.DMA,
            pltpu.SemaphoreType.DMA,
            pltpu.SemaphoreType.DMA,
            pltpu.SemaphoreType.DMA,
            pltpu.SemaphoreType.DMA,
            pltpu.SemaphoreType.DMA,
            pltpu.SemaphoreType.DMA,
        ],
    )
    def seg_k(g_hbm, src2_hbm, dst2_hbm, out_hbm,
              sidx_v, didx_v, b0, b1, b2, acc_sh,
              g0, g1, g2, s0, s1, s2, zsem):
        c = lax.axis_index("c")
        s = lax.axis_index("s")
        bufs = (b0, b1, b2)
        gsems = (g0, g1, g2)
        ssems = (s0, s1, s2)

        def fill_zero(i, carry):
            def inner(j, carry2):
                b0[i, pl.ds(j * 16, 16)] = jnp.zeros((16,), jnp.float32)
                return carry2
            return lax.fori_loop(0, D // 16, inner, carry)

        lax.fori_loop(0, K, fill_zero, 0)

        cbase = c * (EPC // K) + s * NCHUNK
        pltpu.sync_copy(src2_hbm.at[pl.ds(cbase, NCHUNK)], sidx_v)
        pltpu.sync_copy(dst2_hbm.at[pl.ds(cbase, NCHUNK)], didx_v)
        row0 = s * SRPT
        for z in range(SRPT // K):
            pltpu.sync_copy(b0, acc_sh.at[pl.ds(row0 + z * K, K)])
        rem = SRPT - (SRPT // K) * K
        pltpu.sync_copy(b0.at[pl.ds(0, rem)],
                        acc_sh.at[pl.ds(row0 + (SRPT // K) * K, rem)])
        pltpu.sync_copy(b0.at[pl.ds(0, N - NS * SRPT)],
                        acc_sh.at[pl.ds(NS * SRPT, N - NS * SRPT)])
        plsc.subcore_barrier()

        def gather(p, j):
            return pltpu.async_copy(g_hbm.at[sidx_v.at[p]], bufs[j], gsems[j])

        def scatter(p, j):
            return pltpu.async_copy(bufs[j], acc_sh.at[didx_v.at[p]],
                                    ssems[j], add=True)

        def wait_gather(p, j):
            pltpu.make_async_copy(g_hbm.at[sidx_v.at[p]], bufs[j],
                                  gsems[j]).wait()

        def wait_scatter(p, j):
            pltpu.make_async_copy(bufs[j], acc_sh.at[didx_v.at[p]],
                                  ssems[j]).wait()

        gather(0, 0)
        gather(1, 1)
        wait_gather(0, 0)
        scatter(0, 0)
        gather(2, 2)
        wait_gather(1, 1)
        scatter(1, 1)
        wait_scatter(0, 0)
        gather(3, 0)
        wait_gather(2, 2)
        scatter(2, 2)
        wait_scatter(1, 1)
        gather(4, 1)

        def body(q, carry):
            for dp in range(3):
                p = q * 3 + dp
                jn = (dp + 2) % 3
                wait_gather(p, dp)
                scatter(p, dp)
                wait_scatter(p - 1, jn)
                gather(p + 2, jn)
            return carry

        lax.fori_loop(1, (NCHUNK - 2) // 3, body, 0)
        for p in range(NCHUNK - 2, NCHUNK):
            j = p % 3
            wait_gather(p, j)
            scatter(p, j)
            wait_scatter(p - 1, (p - 1) % 3)
        wait_scatter(NCHUNK - 1, (NCHUNK - 1) % 3)

        plsc.subcore_barrier()
        pltpu.sync_copy(acc_sh.at[pl.ds(row0, SRPT)],
                        out_hbm.at[c, pl.ds(row0, SRPT)])
        pltpu.sync_copy(acc_sh.at[pl.ds(NS * SRPT, N - NS * SRPT)],
                        out_hbm.at[c, pl.ds(NS * SRPT, N - NS * SRPT)])

    return seg_k


def _dinv(degp_ref):
    deg = degp_ref[0, :N, :] + degp_ref[1, :N, :]
    return lax.rsqrt(deg[:, 0:1] + 1.0)


def _dot(a, w):
    return jnp.dot(a, w, preferred_element_type=jnp.float32,
                   precision=lax.Precision.HIGHEST)


def _tc_pre(x, W1, degp):
    def body(x_ref, w_ref, degp_ref, out_ref):
        out_ref[...] = _dinv(degp_ref) * _dot(x_ref[...], w_ref[...])

    return pl.pallas_call(
        body,
        out_shape=jax.ShapeDtypeStruct((N, W1.shape[1]), jnp.float32),
    )(x, W1, degp)


def _tc_mid(pp, g, degp, b, Wn):
    def body(pp_ref, g_ref, degp_ref, b_ref, w_ref, out_ref):
        dinv = _dinv(degp_ref)
        h = dinv * (pp_ref[0, :N, :] + pp_ref[1, :N, :] + g_ref[...]) + b_ref[...]
        h = jnp.maximum(h, 0.0)
        out_ref[...] = dinv * _dot(h, w_ref[...])

    return pl.pallas_call(
        body,
        out_shape=jax.ShapeDtypeStruct((N, Wn.shape[1]), jnp.float32),
    )(pp, g, degp, b, Wn)


def _tc_fin(pp, g, degp, b):
    def body(pp_ref, g_ref, degp_ref, b_ref, out_ref):
        dinv = _dinv(degp_ref)
        out_ref[...] = dinv * (pp_ref[0, :N, :] + pp_ref[1, :N, :] + g_ref[...]) + b_ref[...]

    return pl.pallas_call(
        body,
        out_shape=jax.ShapeDtypeStruct((N, g.shape[1]), jnp.float32),
    )(pp, g, degp, b)


def kernel(x, edge_index, W1, b1, W2, b2, W3, b3):
    ei = edge_index.astype(jnp.int32)
    src, dst = ei[0], ei[1]

    src2 = src.reshape(E // K, K)
    dst2 = dst.reshape(E // K, K)
    degp = _deg_partials(dst2)

    g1 = _tc_pre(x, W1, degp)
    pp1 = _make_segsum(128)(g1, src2, dst2)
    g2 = _tc_mid(pp1, g1, degp, b1.reshape(1, -1), W2)
    pp2 = _make_segsum(64)(g2, src2, dst2)
    g3 = _tc_mid(pp2, g2, degp, b2.reshape(1, -1), W3)
    pp3 = _make_segsum(64)(g3, src2, dst2)
    return _tc_fin(pp3, g3, degp, b3.reshape(1, -1))
